# baseline (speedup 1.0000x reference)
import functools

import jax
import jax.numpy as jnp
from jax import lax
from jax.experimental import pallas as pl
from jax.experimental.pallas import tpu as pltpu
from jax.experimental.pallas import tpu_sc as plsc

N = 16384
VOCAB = 1000
HIDDEN = 128

NC, NS = 2, 16
NW = NC * NS
RPW = N // NW

G = 16
NCH = RPW // G
PITCH = 1016
NVEC = 63


def _sc_body(attr_hbm, table_hbm, out_hbm,
             xb0, xb1, tmax, tidx, idxbuf, outbuf, s0, s1, gsem):
    wid = lax.axis_index("s") * NC + lax.axis_index("c")
    base = wid * RPW

    neg = jnp.full((16,), -jnp.inf, jnp.float32)
    lane = lax.broadcasted_iota(jnp.int32, (16,), 0)

    for xb in (xb0, xb1):
        for r in range(G):
            xb[r, pl.ds(1000, 16)] = neg

    def chunk_src(ch):
        return attr_hbm.at[pl.ds(base + ch * G, G), :]

    pltpu.make_async_copy(chunk_src(0), xb0.at[:, pl.ds(0, VOCAB)], s0).start()
    pltpu.make_async_copy(chunk_src(1), xb1.at[:, pl.ds(0, VOCAB)], s1).start()

    def pair(i, carry):
        for b, xb, sem in ((0, xb0, s0), (1, xb1, s1)):
            ch = i * 2 + b
            pltpu.make_async_copy(
                chunk_src(ch), xb.at[:, pl.ds(0, VOCAB)], sem).wait()
            for r in range(G):
                vmax = neg
                vidx = jnp.zeros((16,), jnp.int32)
                for j in range(NVEC):
                    x = xb[r, pl.ds(j * 16, 16)]
                    m = x > vmax
                    vmax = jnp.where(m, x, vmax)
                    vidx = jnp.where(m, jnp.full((16,), j, jnp.int32), vidx)
                tmax[r, pl.ds(0, 16)] = vmax
                tidx[r, pl.ds(0, 16)] = vidx * 16 + lane
            best = neg
            bidx = jnp.zeros((16,), jnp.int32)
            for c in range(16):
                cc = jnp.full((16,), c, jnp.int32)
                v = plsc.load_gather(tmax, [lane, cc])
                iv = plsc.load_gather(tidx, [lane, cc])
                upd = (v > best) | ((v == best) & (iv < bidx))
                best = jnp.where(upd, v, best)
                bidx = jnp.where(upd, iv, bidx)
            idxbuf[pl.ds(ch * G, 16)] = bidx

            nxt = ch + 2

            @pl.when(nxt < NCH)
            def _():
                pltpu.make_async_copy(
                    chunk_src(nxt), xb.at[:, pl.ds(0, VOCAB)], sem).start()
        return carry

    lax.fori_loop(0, NCH // 2, pair, 0)

    pltpu.async_copy(table_hbm.at[idxbuf], outbuf, gsem).wait()
    pltpu.sync_copy(outbuf, out_hbm.at[pl.ds(base, RPW)])


@functools.cache
def _make_sc_kernel():
    mesh = plsc.VectorSubcoreMesh(
        core_axis_name="c", subcore_axis_name="s", num_cores=NC, num_subcores=NS
    )
    return pl.kernel(
        _sc_body,
        out_type=jax.ShapeDtypeStruct((N, HIDDEN), jnp.float32),
        mesh=mesh,
        scratch_types=[
            pltpu.VMEM((G, PITCH), jnp.float32),
            pltpu.VMEM((G, PITCH), jnp.float32),
            pltpu.VMEM((16, 17), jnp.float32),
            pltpu.VMEM((16, 17), jnp.int32),
            pltpu.VMEM((RPW,), jnp.int32),
            pltpu.VMEM((RPW, HIDDEN), jnp.float32),
            pltpu.SemaphoreType.DMA,
            pltpu.SemaphoreType.DMA,
            pltpu.SemaphoreType.DMA,
        ],
        compiler_params=pltpu.CompilerParams(
            use_tc_tiling_on_sc=False, needs_layout_passes=False
        ),
    )


@jax.jit
def kernel(clique_attr, embedding_weight):
    return _make_sc_kernel()(clique_attr, embedding_weight)


# speedup vs baseline: 1.3758x; 1.3758x over previous
import functools

import jax
import jax.numpy as jnp
from jax import lax
from jax.experimental import pallas as pl
from jax.experimental.pallas import tpu as pltpu
from jax.experimental.pallas import tpu_sc as plsc

N = 16384
VOCAB = 1000
HIDDEN = 128

NC, NS = 2, 16
NW = NC * NS
RPW = N // NW

G = 16
RG = 8
NCH = RPW // G
NFULL = 62
TAIL0 = VOCAB - 16


def _sc_body(attr_hbm, table_hbm, out_hbm,
             xb0, xb1, idxbuf, outbuf, s0, s1, gsem):
    wid = lax.axis_index("s") * NC + lax.axis_index("c")
    base = wid * RPW

    lane = lax.broadcasted_iota(jnp.int32, (16,), 0)
    neg = jnp.full((16,), -jnp.inf, jnp.float32)
    tail_col = lane + TAIL0
    big = jnp.full((16,), VOCAB, jnp.int32)

    def chunk_src(ch):
        return attr_hbm.at[pl.ds(base + ch * G, G), :]

    pltpu.make_async_copy(chunk_src(0), xb0, s0).start()
    pltpu.make_async_copy(chunk_src(1), xb1, s1).start()

    def pair(i, carry):
        for b, xb, sem in ((0, xb0, s0), (1, xb1, s1)):
            ch = i * 2 + b
            pltpu.make_async_copy(chunk_src(ch), xb, sem).wait()

            def rowgroup(rg, acc):
                r0 = rg * RG
                vmax = [neg] * RG
                vidx = [lane] * RG
                col = lane
                for j in range(NFULL):
                    for k in range(RG):
                        x = xb[r0 + k, pl.ds(j * 16, 16)]
                        m = x > vmax[k]
                        vmax[k] = jnp.where(m, x, vmax[k])
                        vidx[k] = jnp.where(m, col, vidx[k])
                    col = col + 16
                for k in range(RG):
                    x = xb[r0 + k, pl.ds(TAIL0, 16)]
                    m = x > vmax[k]
                    vmax[k] = jnp.where(m, x, vmax[k])
                    vidx[k] = jnp.where(m, tail_col, vidx[k])
                for k in range(RG):
                    m0 = jnp.max(vmax[k])
                    cand = jnp.where(vmax[k] == m0, vidx[k], big)
                    acc = jnp.where(lane == r0 + k, jnp.min(cand), acc)
                return acc

            acc = lax.fori_loop(0, G // RG, rowgroup, lane)
            idxbuf[pl.ds(ch * G, 16)] = acc

            nxt = ch + 2

            @pl.when(nxt < NCH)
            def _():
                pltpu.make_async_copy(chunk_src(nxt), xb, sem).start()
        return carry

    lax.fori_loop(0, NCH // 2, pair, 0)

    pltpu.async_copy(table_hbm.at[idxbuf], outbuf, gsem).wait()
    pltpu.sync_copy(outbuf, out_hbm.at[pl.ds(base, RPW)])


@functools.cache
def _make_sc_kernel():
    mesh = plsc.VectorSubcoreMesh(
        core_axis_name="c", subcore_axis_name="s", num_cores=NC, num_subcores=NS
    )
    return pl.kernel(
        _sc_body,
        out_type=jax.ShapeDtypeStruct((N, HIDDEN), jnp.float32),
        mesh=mesh,
        scratch_types=[
            pltpu.VMEM((G, VOCAB), jnp.float32),
            pltpu.VMEM((G, VOCAB), jnp.float32),
            pltpu.VMEM((RPW,), jnp.int32),
            pltpu.VMEM((RPW, HIDDEN), jnp.float32),
            pltpu.SemaphoreType.DMA,
            pltpu.SemaphoreType.DMA,
            pltpu.SemaphoreType.DMA,
        ],
        compiler_params=pltpu.CompilerParams(needs_layout_passes=False),
    )


@jax.jit
def kernel(clique_attr, embedding_weight):
    return _make_sc_kernel()(clique_attr, embedding_weight)


# speedup vs baseline: 1.6752x; 1.2176x over previous
import functools

import jax
import jax.numpy as jnp
from jax import lax
from jax.experimental import pallas as pl
from jax.experimental.pallas import tpu as pltpu
from jax.experimental.pallas import tpu_sc as plsc

N = 16384
VOCAB = 1000
HIDDEN = 128

NC, NS = 2, 16
NW = NC * NS
RPW = N // NW

G = 8
RG = 4
NBUF = 4
NCH = RPW // G
NFULL = 62
TAIL0 = VOCAB - 16


def _sc_body(attr_hbm, table_hbm, out_hbm,
             xb0, xb1, xb2, xb3, idxbuf, outbuf, s0, s1, s2, s3, gsem):
    bufs = (xb0, xb1, xb2, xb3)
    sems = (s0, s1, s2, s3)
    wid = lax.axis_index("s") * NC + lax.axis_index("c")
    base = wid * RPW

    lane = lax.broadcasted_iota(jnp.int32, (16,), 0)
    neg = jnp.full((16,), -jnp.inf, jnp.float32)
    tail_col = lane + TAIL0
    big = jnp.full((16,), VOCAB, jnp.int32)

    def chunk_src(ch):
        return attr_hbm.at[pl.ds(base + ch * G, G), :]

    for b in range(NBUF):
        pltpu.make_async_copy(chunk_src(b), bufs[b], sems[b]).start()

    def ring(i, carry):
        acc = lane
        for b in range(NBUF):
            xb, sem = bufs[b], sems[b]
            ch = i * NBUF + b
            half = (b % 2) * G
            pltpu.make_async_copy(chunk_src(ch), xb, sem).wait()

            def rowgroup(rg, acc, xb=xb, half=half):
                r0 = rg * RG
                vmax = [neg] * RG
                vidx = [lane] * RG
                col = lane
                for j in range(NFULL):
                    for k in range(RG):
                        x = xb[r0 + k, pl.ds(j * 16, 16)]
                        m = x > vmax[k]
                        vmax[k] = jnp.where(m, x, vmax[k])
                        vidx[k] = jnp.where(m, col, vidx[k])
                    col = col + 16
                for k in range(RG):
                    x = xb[r0 + k, pl.ds(TAIL0, 16)]
                    m = x > vmax[k]
                    vmax[k] = jnp.where(m, x, vmax[k])
                    vidx[k] = jnp.where(m, tail_col, vidx[k])
                for k in range(RG):
                    m0 = jnp.max(vmax[k])
                    cand = jnp.where(vmax[k] == m0, vidx[k], big)
                    acc = jnp.where(lane == half + r0 + k, jnp.min(cand), acc)
                return acc

            acc = lax.fori_loop(0, G // RG, rowgroup, acc)
            if b % 2 == 1:
                idxbuf[pl.ds((ch - 1) * G, 16)] = acc
                acc = lane

            nxt = ch + NBUF

            @pl.when(nxt < NCH)
            def _(xb=xb, sem=sem, nxt=nxt):
                pltpu.make_async_copy(chunk_src(nxt), xb, sem).start()
        return carry

    lax.fori_loop(0, NCH // NBUF, ring, 0)

    pltpu.async_copy(table_hbm.at[idxbuf], outbuf, gsem).wait()
    pltpu.sync_copy(outbuf, out_hbm.at[pl.ds(base, RPW)])


@functools.cache
def _make_sc_kernel():
    mesh = plsc.VectorSubcoreMesh(
        core_axis_name="c", subcore_axis_name="s", num_cores=NC, num_subcores=NS
    )
    return pl.kernel(
        _sc_body,
        out_type=jax.ShapeDtypeStruct((N, HIDDEN), jnp.float32),
        mesh=mesh,
        scratch_types=[
            pltpu.VMEM((G, VOCAB), jnp.float32),
            pltpu.VMEM((G, VOCAB), jnp.float32),
            pltpu.VMEM((G, VOCAB), jnp.float32),
            pltpu.VMEM((G, VOCAB), jnp.float32),
            pltpu.VMEM((RPW,), jnp.int32),
            pltpu.VMEM((RPW, HIDDEN), jnp.float32),
            pltpu.SemaphoreType.DMA,
            pltpu.SemaphoreType.DMA,
            pltpu.SemaphoreType.DMA,
            pltpu.SemaphoreType.DMA,
            pltpu.SemaphoreType.DMA,
        ],
        compiler_params=pltpu.CompilerParams(needs_layout_passes=False),
    )


@jax.jit
def kernel(clique_attr, embedding_weight):
    return _make_sc_kernel()(clique_attr, embedding_weight)


# speedup vs baseline: 2.4171x; 1.4428x over previous
import functools

import jax
import jax.numpy as jnp
from jax import lax
from jax.experimental import pallas as pl
from jax.experimental.pallas import tpu as pltpu
from jax.experimental.pallas import tpu_sc as plsc

N = 16384
VOCAB = 1000
HIDDEN = 128

BR = 2048
WPAD = 1024

NC, NS = 2, 16
NW = NC * NS
BPW = N // NW


def _argmax_body(x_ref, idx_ref):
    x = x_ref[...]
    tail = x[:, 896:WPAD]
    tcol = lax.broadcasted_iota(jnp.int32, tail.shape, 1) + 896
    tail = jnp.where(tcol < VOCAB, tail, -jnp.inf)
    xm = jnp.concatenate([x[:, :896], tail], axis=1)
    m0 = jnp.max(xm, axis=1, keepdims=True)
    col = lax.broadcasted_iota(jnp.int32, xm.shape, 1)
    cand = jnp.where(xm == m0, col, WPAD)
    idx_ref[...] = jnp.min(cand, axis=1)


def _tc_argmax(clique_attr):
    return pl.pallas_call(
        _argmax_body,
        grid=(N // BR,),
        in_specs=[pl.BlockSpec((BR, WPAD), lambda i: (i, 0))],
        out_specs=pl.BlockSpec((BR,), lambda i: (i,)),
        out_shape=jax.ShapeDtypeStruct((N,), jnp.int32),
    )(clique_attr)


def _sc_gather_body(table_hbm, idx_hbm, out_hbm, idx_v, rows_v, gsem):
    wid = lax.axis_index("s") * NC + lax.axis_index("c")
    base = wid * BPW
    pltpu.sync_copy(idx_hbm.at[pl.ds(base, BPW)], idx_v)
    pltpu.async_copy(table_hbm.at[idx_v], rows_v, gsem).wait()
    pltpu.sync_copy(rows_v, out_hbm.at[pl.ds(base, BPW)])


@functools.cache
def _make_sc_gather():
    mesh = plsc.VectorSubcoreMesh(
        core_axis_name="c", subcore_axis_name="s", num_cores=NC, num_subcores=NS
    )
    return pl.kernel(
        _sc_gather_body,
        out_type=jax.ShapeDtypeStruct((N, HIDDEN), jnp.float32),
        mesh=mesh,
        scratch_types=[
            pltpu.VMEM((BPW,), jnp.int32),
            pltpu.VMEM((BPW, HIDDEN), jnp.float32),
            pltpu.SemaphoreType.DMA,
        ],
    )


@jax.jit
def kernel(clique_attr, embedding_weight):
    idx = _tc_argmax(clique_attr)
    return _make_sc_gather()(embedding_weight, idx)
